# Initial kernel scaffold; baseline (speedup 1.0000x reference)
import jax
import jax.numpy as jnp
from jax.experimental import pallas as pl


def kernel(tokens, table):
    raise NotImplementedError("write your pallas kernel here")


# speedup vs baseline: 3.3339x; 3.3339x over previous
import jax
import jax.numpy as jnp
from jax import lax
from jax.experimental import pallas as pl
from jax.experimental.pallas import tpu as pltpu
from jax.experimental.pallas import tpu_sc as plsc

VOCAB_E = 100000
EMBED_E = 128
B_E = 4096
L_E = 50

NC = 2
NS = 16
NW = NC * NS

N_TOK = B_E * L_E
PER_W = N_TOK // NW
CH = 400
NCHUNK = PER_W // CH


def _emb_body(tokens_hbm, table_hbm, out_hbm, idx0, idx1, rows0, rows1,
              isem0, isem1, gsem0, gsem1, osem0, osem1):
  wid = lax.axis_index("s") * NC + lax.axis_index("c")
  base = wid * PER_W

  idx = (idx0, idx1)
  rows = (rows0, rows1)
  isem = (isem0, isem1)
  gsem = (gsem0, gsem1)
  osem = (osem0, osem1)

  pltpu.async_copy(tokens_hbm.at[wid, 0], idx0, isem0)
  pltpu.async_copy(tokens_hbm.at[wid, 1], idx1, isem1)

  for c in range(NCHUNK):
    b = c % 2
    pltpu.make_async_copy(tokens_hbm.at[wid, c], idx[b], isem[b]).wait()
    if c >= 2:
      pltpu.make_async_copy(rows[b], out_hbm.at[pl.ds(0, CH)], osem[b]).wait()
    pltpu.async_copy(table_hbm.at[idx[b]], rows[b], gsem[b]).wait()
    if c + 2 < NCHUNK:
      pltpu.async_copy(tokens_hbm.at[wid, c + 2], idx[b], isem[b])
    pltpu.async_copy(rows[b], out_hbm.at[pl.ds(base + c * CH, CH)], osem[b])

  pltpu.make_async_copy(rows0, out_hbm.at[pl.ds(0, CH)], osem0).wait()
  pltpu.make_async_copy(rows1, out_hbm.at[pl.ds(0, CH)], osem1).wait()


@jax.jit
def _embed(tokens_flat, table):
  k = pl.kernel(
      _emb_body,
      out_type=jax.ShapeDtypeStruct((N_TOK, EMBED_E), jnp.float32),
      mesh=plsc.VectorSubcoreMesh(core_axis_name="c", subcore_axis_name="s"),
      scratch_types=[
          pltpu.VMEM((CH,), jnp.int32),
          pltpu.VMEM((CH,), jnp.int32),
          pltpu.VMEM((CH, EMBED_E), jnp.float32),
          pltpu.VMEM((CH, EMBED_E), jnp.float32),
          pltpu.SemaphoreType.DMA,
          pltpu.SemaphoreType.DMA,
          pltpu.SemaphoreType.DMA,
          pltpu.SemaphoreType.DMA,
          pltpu.SemaphoreType.DMA,
          pltpu.SemaphoreType.DMA,
      ],
  )
  return k(tokens_flat, table)


def kernel(tokens, table):
  tokens_flat = tokens.astype(jnp.int32).reshape(NW, NCHUNK, CH)
  out = _embed(tokens_flat, table)
  return out.reshape(B_E, L_E, EMBED_E)
